# baseline (speedup 1.0000x reference)
import jax
import jax.numpy as jnp
from jax import lax
from jax.experimental import pallas as pl
from jax.experimental.pallas import tpu as pltpu
from jax.experimental.pallas import tpu_sc as plsc

_N = 10000
_E = 320000
_G = 64
_H = 32
_F = 128
_NC = 2
_NS = 16
_NW = _NC * _NS
_CHUNK = 128
_NCH = 80
_EPW = _NCH * _CHUNK
_EPAD = _NW * _EPW
_RPS = 640
_NP = _NS * _RPS

_F32 = jnp.float32
_HIGH = lax.Precision.HIGHEST


def _mesh():
    return plsc.VectorSubcoreMesh(core_axis_name="c", subcore_axis_name="s")


def _deg_body(dst_hbm, ones_hbm, zrow_hbm, out_hbm, dst_v, ones_v, acc_sh,
              ssem):
    cid = lax.axis_index("c")
    sid = lax.axis_index("s")
    wid = sid * _NC + cid
    base = sid * _RPS
    pltpu.sync_copy(zrow_hbm, acc_sh.at[pl.ds(base, _RPS)])
    pltpu.sync_copy(ones_hbm, ones_v)
    pltpu.sync_copy(dst_hbm.at[wid], dst_v)
    plsc.subcore_barrier()

    def step(j, carry):
        pltpu.async_copy(ones_v, acc_sh.at[dst_v.at[j]], ssem, add=True)

        @pl.when(j >= 8)
        def _():
            pltpu.make_async_copy(ones_v, acc_sh.at[dst_v.at[j]], ssem).wait()

        return carry

    lax.fori_loop(0, _NCH, step, 0)

    def drain(j, carry):
        pltpu.make_async_copy(ones_v, acc_sh.at[dst_v.at[j]], ssem).wait()
        return carry

    lax.fori_loop(0, 8, drain, 0)
    plsc.subcore_barrier()
    pltpu.sync_copy(acc_sh.at[pl.ds(base, _RPS)],
                    out_hbm.at[cid].at[pl.ds(base, _RPS)])


_deg_kernel = pl.kernel(
    _deg_body,
    out_type=jax.ShapeDtypeStruct((_NC, _NP), _F32),
    mesh=_mesh(),
    scratch_types=[
        pltpu.VMEM((_NCH, _CHUNK), jnp.int32),
        pltpu.VMEM((_CHUNK,), _F32),
        pltpu.VMEM_SHARED((_NP,), _F32),
        pltpu.SemaphoreType.DMA,
    ],
)


def _make_prop(width, nbuf, look, split_features, chunk=_CHUNK):
    nblk = _NC if split_features else 1
    nch = _EPW // chunk
    tch = nblk * nch

    def body(u_hbm, src_hbm, dst_hbm, zrow_hbm, out_hbm,
             src_v, dst_v, rows_v, acc_sh, *sems):
        gsem = sems[:nbuf]
        ssem = sems[nbuf:]
        cid = lax.axis_index("c")
        sid = lax.axis_index("s")
        base = sid * _RPS
        pltpu.sync_copy(zrow_hbm, acc_sh.at[pl.ds(base, _RPS)])
        if split_features:
            uref = u_hbm.at[cid]
            for h in range(nblk):
                blk = sid * _NC + h
                pltpu.sync_copy(src_hbm.at[blk],
                                src_v.at[pl.ds(h * nch, nch)])
                pltpu.sync_copy(dst_hbm.at[blk],
                                dst_v.at[pl.ds(h * nch, nch)])
        else:
            uref = u_hbm
            wid = sid * _NC + cid
            pltpu.sync_copy(src_hbm.at[wid], src_v)
            pltpu.sync_copy(dst_hbm.at[wid], dst_v)
        plsc.subcore_barrier()

        def gath(j, b):
            return pltpu.async_copy(uref.at[src_v.at[j]], rows_v.at[b],
                                    gsem[b])

        for j in range(look):
            gath(j, j)

        def step(i, carry):
            for b in range(nbuf):
                jj = nbuf * i + b
                tb = (b + look) % nbuf
                pltpu.make_async_copy(uref.at[src_v.at[jj]], rows_v.at[b],
                                      gsem[b]).wait()
                pltpu.async_copy(rows_v.at[b], acc_sh.at[dst_v.at[jj]],
                                 ssem[b], add=True)
                tgt = jj + look

                @pl.when(tgt < tch)
                def _():
                    @pl.when(tgt >= nbuf)
                    def _():
                        pltpu.make_async_copy(
                            rows_v.at[tb], acc_sh.at[dst_v.at[0]],
                            ssem[tb]).wait()

                    gath(tgt, tb)
            return carry

        lax.fori_loop(0, tch // nbuf, step, 0)
        for b in range(nbuf):
            pltpu.make_async_copy(rows_v.at[b], acc_sh.at[dst_v.at[0]],
                                  ssem[b]).wait()
        plsc.subcore_barrier()
        pltpu.sync_copy(acc_sh.at[pl.ds(base, _RPS)],
                        out_hbm.at[cid].at[pl.ds(base, _RPS)])

    return pl.kernel(
        body,
        out_type=jax.ShapeDtypeStruct((_NC, _NP, width), _F32),
        mesh=_mesh(),
        compiler_params=pltpu.CompilerParams(use_tc_tiling_on_sc=False),
        scratch_types=[
            pltpu.VMEM((tch, chunk), jnp.int32),
            pltpu.VMEM((tch, chunk), jnp.int32),
            pltpu.VMEM((nbuf, chunk, width), _F32),
            pltpu.VMEM_SHARED((_NP, width), _F32),
        ] + [pltpu.SemaphoreType.DMA] * (2 * nbuf),
    )


_PCH = 128
_SCH = 512
_prop128 = _make_prop(_F // 2, 4, 3, True, chunk=_PCH)
_prop32 = _make_prop(_H, 4, 3, False, chunk=_SCH)


def _zpad(ref, val, width):
    ref[0:_N, :] = val
    ref[_N:, :] = jnp.zeros((_NP - _N, width), _F32)


def _zpad_halves(ref, val):
    hw = _F // 2
    ref[0, 0:_N, :] = val[:, 0:hw]
    ref[1, 0:_N, :] = val[:, hw:_F]
    ref[0, _N:, :] = jnp.zeros((_NP - _N, hw), _F32)
    ref[1, _N:, :] = jnp.zeros((_NP - _N, hw), _F32)


def _tca_body(degT_ref, x_ref, dinv_ref, u0_ref):
    deg = degT_ref[:, 0:1] + degT_ref[:, 1:2]
    dinv = jnp.where(deg > 0.0, lax.rsqrt(jnp.maximum(deg, 1e-12)), 0.0)
    rows = lax.broadcasted_iota(jnp.int32, (_NP, 1), 0)
    dinv = jnp.where(rows < _N, dinv, 0.0)
    dinv_ref[...] = dinv
    _zpad_halves(u0_ref, dinv[0:_N] * x_ref[...])


_tca = pl.pallas_call(
    _tca_body,
    out_shape=[
        jax.ShapeDtypeStruct((_NP, 1), _F32),
        jax.ShapeDtypeStruct((_NC, _NP, _F // 2), _F32),
    ],
)


def _tcb_body(q_ref, dinv_ref, x_ref, w1_ref, s01_ref, u1_ref):
    s = jnp.concatenate([q_ref[0], q_ref[1]], axis=1)
    dinv = dinv_ref[...]
    tx1 = (-dinv * s)[0:_N]
    s01_ref[...] = x_ref[...] @ w1_ref[0] + tx1 @ w1_ref[1]
    _zpad_halves(u1_ref, dinv[0:_N] * tx1)


_tcb = pl.pallas_call(
    _tcb_body,
    out_shape=[
        jax.ShapeDtypeStruct((_N, _H), _F32),
        jax.ShapeDtypeStruct((_NC, _NP, _F // 2), _F32),
    ],
)


def _tcc_body(q_ref, dinv_ref, x_ref, s01_ref, b1_ref, w1_ref,
              h1_ref, uh1_ref):
    s = jnp.concatenate([q_ref[0], q_ref[1]], axis=1)
    dinv = dinv_ref[...]
    p = (-dinv * s)[0:_N]
    tx2 = 2.0 * p - x_ref[...]
    h1 = jnp.maximum((s01_ref[...] + tx2 @ w1_ref[2]) + b1_ref[...], 0.0)
    h1_ref[...] = h1
    _zpad(uh1_ref, dinv[0:_N] * h1, _H)


_tcc = pl.pallas_call(
    _tcc_body,
    out_shape=[
        jax.ShapeDtypeStruct((_N, _H), _F32),
        jax.ShapeDtypeStruct((_NP, _H), _F32),
    ],
)


def _tcd_body(q_ref, dinv_ref, h_ref, w_ref, s01_ref, ut_ref):
    s = q_ref[0] + q_ref[1]
    dinv = dinv_ref[...]
    t1 = (-dinv * s)[0:_N]
    s01_ref[...] = h_ref[...] @ w_ref[0] + t1 @ w_ref[1]
    _zpad(ut_ref, dinv[0:_N] * t1, _H)


_tcd = pl.pallas_call(
    _tcd_body,
    out_shape=[
        jax.ShapeDtypeStruct((_N, _H), _F32),
        jax.ShapeDtypeStruct((_NP, _H), _F32),
    ],
)


def _tce_body(q_ref, dinv_ref, h_ref, s01_ref, b_ref, w_ref,
              hn_ref, uhn_ref):
    s = q_ref[0] + q_ref[1]
    dinv = dinv_ref[...]
    p = (-dinv * s)[0:_N]
    tx2 = 2.0 * p - h_ref[...]
    hn = jnp.maximum((s01_ref[...] + tx2 @ w_ref[2]) + b_ref[...], 0.0)
    hn_ref[...] = hn
    _zpad(uhn_ref, dinv[0:_N] * hn, _H)


_tce = pl.pallas_call(
    _tce_body,
    out_shape=[
        jax.ShapeDtypeStruct((_N, _H), _F32),
        jax.ShapeDtypeStruct((_NP, _H), _F32),
    ],
)


def _tcf_body(q_ref, dinv_ref, h_ref, s01_ref, b3_ref, w3_ref, batch_ref,
              wf1_ref, bf1_ref, wf2_ref, bf2_ref, out_ref):
    s = q_ref[0] + q_ref[1]
    dinv = dinv_ref[...]
    p = (-dinv * s)[0:_N]
    tx2 = 2.0 * p - h_ref[...]
    h3 = jnp.maximum((s01_ref[...] + tx2 @ w3_ref[2]) + b3_ref[...], 0.0)
    m = (batch_ref[...] ==
         lax.broadcasted_iota(jnp.int32, (_N, _G), 1)).astype(_F32)
    dims = (((0,), (0,)), ((), ()))
    sums = lax.dot_general(m, h3, dims, preferred_element_type=_F32,
                           precision=_HIGH)
    cnt = lax.dot_general(m, jnp.ones((_N, 1), _F32), dims,
                          preferred_element_type=_F32, precision=_HIGH)
    pooled = sums / jnp.maximum(cnt, 1.0)
    r = jnp.maximum(pooled @ wf1_ref[...] + bf1_ref[...], 0.0)
    out_ref[...] = r @ wf2_ref[...] + bf2_ref[...]


_tcf = pl.pallas_call(
    _tcf_body,
    out_shape=jax.ShapeDtypeStruct((_G, 1), _F32),
)


def kernel(x, edge_index, batch, W1, b1, W2, b2, W3, b3, Wf1, bf1, Wf2, bf2):
    pad = jnp.full((_EPAD - _E,), _N, jnp.int32)
    srcf = jnp.concatenate([edge_index[0], pad])
    dstf = jnp.concatenate([edge_index[1], pad])
    srcw = srcf.reshape(_NW, _NCH, _CHUNK)
    dstw = dstf.reshape(_NW, _NCH, _CHUNK)
    srcp = srcf.reshape(_NW, _EPW // _PCH, _PCH)
    dstp = dstf.reshape(_NW, _EPW // _PCH, _PCH)
    srcs = srcf.reshape(_NW, _EPW // _SCH, _SCH)
    dsts = dstf.reshape(_NW, _EPW // _SCH, _SCH)

    ones_c = jnp.ones((_CHUNK,), _F32)
    zrow1 = jnp.zeros((_RPS,), _F32)
    zrow32 = jnp.zeros((_RPS, _H), _F32)
    zrow64 = jnp.zeros((_RPS, _F // 2), _F32)

    degp = _deg_kernel(dstw, ones_c, zrow1)
    dinv, u0 = _tca(degp.T, x)
    qa = _prop128(u0, srcp, dstp, zrow64)
    s01, u1 = _tcb(qa, dinv, x, W1)
    qb = _prop128(u1, srcp, dstp, zrow64)
    h1, uh1 = _tcc(qb, dinv, x, s01, b1.reshape(1, _H), W1)

    q1 = _prop32(uh1, srcs, dsts, zrow32)
    s01_2, ut1 = _tcd(q1, dinv, h1, W2)
    q2 = _prop32(ut1, srcs, dsts, zrow32)
    h2, uh2 = _tce(q2, dinv, h1, s01_2, b2.reshape(1, _H), W2)

    q3 = _prop32(uh2, srcs, dsts, zrow32)
    s01_3, ut2 = _tcd(q3, dinv, h2, W3)
    q4 = _prop32(ut2, srcs, dsts, zrow32)
    out = _tcf(q4, dinv, h2, s01_3, b3.reshape(1, _H), W3,
               batch.reshape(_N, 1), Wf1, bf1.reshape(1, _H),
               Wf2, bf2.reshape(1, 1))
    return out


# speedup vs baseline: 1.3863x; 1.3863x over previous
import jax
import jax.numpy as jnp
from jax import lax
from jax.experimental import pallas as pl
from jax.experimental.pallas import tpu as pltpu
from jax.experimental.pallas import tpu_sc as plsc

_N = 10000
_E = 320000
_G = 64
_H = 32
_F = 128
_NC = 2
_NS = 16
_NW = _NC * _NS
_CHUNK = 128
_NCH = 80
_EPW = _NCH * _CHUNK
_EPAD = _NW * _EPW
_RPS = 640
_NP = _NS * _RPS

_F32 = jnp.float32
_HIGH = lax.Precision.HIGHEST


def _mesh():
    return plsc.VectorSubcoreMesh(core_axis_name="c", subcore_axis_name="s")


def _deg_body(dst_hbm, ones_hbm, zrow_hbm, out_hbm, dst_v, ones_v, acc_sh,
              ssem):
    cid = lax.axis_index("c")
    sid = lax.axis_index("s")
    wid = sid * _NC + cid
    base = sid * _RPS
    pltpu.sync_copy(zrow_hbm, acc_sh.at[pl.ds(base, _RPS)])
    pltpu.sync_copy(ones_hbm, ones_v)
    pltpu.sync_copy(dst_hbm.at[wid], dst_v)
    plsc.subcore_barrier()

    def step(j, carry):
        pltpu.async_copy(ones_v, acc_sh.at[dst_v.at[j]], ssem, add=True)

        @pl.when(j >= 8)
        def _():
            pltpu.make_async_copy(ones_v, acc_sh.at[dst_v.at[j]], ssem).wait()

        return carry

    lax.fori_loop(0, _NCH, step, 0)

    def drain(j, carry):
        pltpu.make_async_copy(ones_v, acc_sh.at[dst_v.at[j]], ssem).wait()
        return carry

    lax.fori_loop(0, 8, drain, 0)
    plsc.subcore_barrier()
    pltpu.sync_copy(acc_sh.at[pl.ds(base, _RPS)],
                    out_hbm.at[cid].at[pl.ds(base, _RPS)])


_deg_kernel = pl.kernel(
    _deg_body,
    out_type=jax.ShapeDtypeStruct((_NC, _NP), _F32),
    mesh=_mesh(),
    scratch_types=[
        pltpu.VMEM((_NCH, _CHUNK), jnp.int32),
        pltpu.VMEM((_CHUNK,), _F32),
        pltpu.VMEM_SHARED((_NP,), _F32),
        pltpu.SemaphoreType.DMA,
    ],
)


def _make_prop(width, nbuf, look, split_features, chunk=_CHUNK):
    nblk = _NC if split_features else 1
    nch = _EPW // chunk
    tch = nblk * nch

    def body(u_hbm, src_hbm, dst_hbm, zrow_hbm, out_hbm,
             src_v, dst_v, rows_v, acc_sh, *rest):
        if split_features:
            sems = rest
        else:
            utab_sh = rest[0]
            sems = rest[1:]
        gsem = sems[:nbuf]
        ssem = sems[nbuf:]
        cid = lax.axis_index("c")
        sid = lax.axis_index("s")
        base = sid * _RPS
        pltpu.sync_copy(zrow_hbm, acc_sh.at[pl.ds(base, _RPS)])
        if split_features:
            uref = u_hbm.at[cid]
            for h in range(nblk):
                blk = sid * _NC + h
                pltpu.sync_copy(src_hbm.at[blk],
                                src_v.at[pl.ds(h * nch, nch)])
                pltpu.sync_copy(dst_hbm.at[blk],
                                dst_v.at[pl.ds(h * nch, nch)])
        else:
            uref = utab_sh
            pltpu.sync_copy(u_hbm.at[pl.ds(base, _RPS)],
                            utab_sh.at[pl.ds(base, _RPS)])
            wid = sid * _NC + cid
            pltpu.sync_copy(src_hbm.at[wid], src_v)
            pltpu.sync_copy(dst_hbm.at[wid], dst_v)
        plsc.subcore_barrier()

        def gath(j, b):
            return pltpu.async_copy(uref.at[src_v.at[j]], rows_v.at[b],
                                    gsem[b])

        for j in range(look):
            gath(j, j)

        def step(i, carry):
            for b in range(nbuf):
                jj = nbuf * i + b
                tb = (b + look) % nbuf
                pltpu.make_async_copy(uref.at[src_v.at[jj]], rows_v.at[b],
                                      gsem[b]).wait()
                pltpu.async_copy(rows_v.at[b], acc_sh.at[dst_v.at[jj]],
                                 ssem[b], add=True)
                tgt = jj + look

                @pl.when(tgt < tch)
                def _():
                    @pl.when(tgt >= nbuf)
                    def _():
                        pltpu.make_async_copy(
                            rows_v.at[tb], acc_sh.at[dst_v.at[0]],
                            ssem[tb]).wait()

                    gath(tgt, tb)
            return carry

        lax.fori_loop(0, tch // nbuf, step, 0)
        for b in range(nbuf):
            pltpu.make_async_copy(rows_v.at[b], acc_sh.at[dst_v.at[0]],
                                  ssem[b]).wait()
        plsc.subcore_barrier()
        pltpu.sync_copy(acc_sh.at[pl.ds(base, _RPS)],
                        out_hbm.at[cid].at[pl.ds(base, _RPS)])

    return pl.kernel(
        body,
        out_type=jax.ShapeDtypeStruct((_NC, _NP, width), _F32),
        mesh=_mesh(),
        compiler_params=pltpu.CompilerParams(use_tc_tiling_on_sc=False),
        scratch_types=[
            pltpu.VMEM((tch, chunk), jnp.int32),
            pltpu.VMEM((tch, chunk), jnp.int32),
            pltpu.VMEM((nbuf, chunk, width), _F32),
            pltpu.VMEM_SHARED((_NP, width), _F32),
        ] + ([] if split_features else [pltpu.VMEM_SHARED((_NP, width), _F32)])
          + [pltpu.SemaphoreType.DMA] * (2 * nbuf),
    )


_PCH = 128
_SCH = 512
_prop128 = _make_prop(_F // 2, 4, 3, True, chunk=_PCH)
_prop32 = _make_prop(_H, 4, 3, False, chunk=_SCH)


def _zpad(ref, val, width):
    ref[0:_N, :] = val
    ref[_N:, :] = jnp.zeros((_NP - _N, width), _F32)


def _zpad_halves(ref, val):
    hw = _F // 2
    ref[0, 0:_N, :] = val[:, 0:hw]
    ref[1, 0:_N, :] = val[:, hw:_F]
    ref[0, _N:, :] = jnp.zeros((_NP - _N, hw), _F32)
    ref[1, _N:, :] = jnp.zeros((_NP - _N, hw), _F32)


def _tca_body(degT_ref, x_ref, dinv_ref, u0_ref):
    deg = degT_ref[:, 0:1] + degT_ref[:, 1:2]
    dinv = jnp.where(deg > 0.0, lax.rsqrt(jnp.maximum(deg, 1e-12)), 0.0)
    rows = lax.broadcasted_iota(jnp.int32, (_NP, 1), 0)
    dinv = jnp.where(rows < _N, dinv, 0.0)
    dinv_ref[...] = dinv
    _zpad_halves(u0_ref, dinv[0:_N] * x_ref[...])


_tca = pl.pallas_call(
    _tca_body,
    out_shape=[
        jax.ShapeDtypeStruct((_NP, 1), _F32),
        jax.ShapeDtypeStruct((_NC, _NP, _F // 2), _F32),
    ],
)


def _tcb_body(q_ref, dinv_ref, x_ref, w1_ref, s01_ref, u1_ref):
    s = jnp.concatenate([q_ref[0], q_ref[1]], axis=1)
    dinv = dinv_ref[...]
    tx1 = (-dinv * s)[0:_N]
    s01_ref[...] = x_ref[...] @ w1_ref[0] + tx1 @ w1_ref[1]
    _zpad_halves(u1_ref, dinv[0:_N] * tx1)


_tcb = pl.pallas_call(
    _tcb_body,
    out_shape=[
        jax.ShapeDtypeStruct((_N, _H), _F32),
        jax.ShapeDtypeStruct((_NC, _NP, _F // 2), _F32),
    ],
)


def _tcc_body(q_ref, dinv_ref, x_ref, s01_ref, b1_ref, w1_ref,
              h1_ref, uh1_ref):
    s = jnp.concatenate([q_ref[0], q_ref[1]], axis=1)
    dinv = dinv_ref[...]
    p = (-dinv * s)[0:_N]
    tx2 = 2.0 * p - x_ref[...]
    h1 = jnp.maximum((s01_ref[...] + tx2 @ w1_ref[2]) + b1_ref[...], 0.0)
    h1_ref[...] = h1
    _zpad(uh1_ref, dinv[0:_N] * h1, _H)


_tcc = pl.pallas_call(
    _tcc_body,
    out_shape=[
        jax.ShapeDtypeStruct((_N, _H), _F32),
        jax.ShapeDtypeStruct((_NP, _H), _F32),
    ],
)


def _tcd_body(q_ref, dinv_ref, h_ref, w_ref, s01_ref, ut_ref):
    s = q_ref[0] + q_ref[1]
    dinv = dinv_ref[...]
    t1 = (-dinv * s)[0:_N]
    s01_ref[...] = h_ref[...] @ w_ref[0] + t1 @ w_ref[1]
    _zpad(ut_ref, dinv[0:_N] * t1, _H)


_tcd = pl.pallas_call(
    _tcd_body,
    out_shape=[
        jax.ShapeDtypeStruct((_N, _H), _F32),
        jax.ShapeDtypeStruct((_NP, _H), _F32),
    ],
)


def _tce_body(q_ref, dinv_ref, h_ref, s01_ref, b_ref, w_ref,
              hn_ref, uhn_ref):
    s = q_ref[0] + q_ref[1]
    dinv = dinv_ref[...]
    p = (-dinv * s)[0:_N]
    tx2 = 2.0 * p - h_ref[...]
    hn = jnp.maximum((s01_ref[...] + tx2 @ w_ref[2]) + b_ref[...], 0.0)
    hn_ref[...] = hn
    _zpad(uhn_ref, dinv[0:_N] * hn, _H)


_tce = pl.pallas_call(
    _tce_body,
    out_shape=[
        jax.ShapeDtypeStruct((_N, _H), _F32),
        jax.ShapeDtypeStruct((_NP, _H), _F32),
    ],
)


def _tcf_body(q_ref, dinv_ref, h_ref, s01_ref, b3_ref, w3_ref, batch_ref,
              wf1_ref, bf1_ref, wf2_ref, bf2_ref, out_ref):
    s = q_ref[0] + q_ref[1]
    dinv = dinv_ref[...]
    p = (-dinv * s)[0:_N]
    tx2 = 2.0 * p - h_ref[...]
    h3 = jnp.maximum((s01_ref[...] + tx2 @ w3_ref[2]) + b3_ref[...], 0.0)
    m = (batch_ref[...] ==
         lax.broadcasted_iota(jnp.int32, (_N, _G), 1)).astype(_F32)
    dims = (((0,), (0,)), ((), ()))
    sums = lax.dot_general(m, h3, dims, preferred_element_type=_F32,
                           precision=_HIGH)
    cnt = lax.dot_general(m, jnp.ones((_N, 1), _F32), dims,
                          preferred_element_type=_F32, precision=_HIGH)
    pooled = sums / jnp.maximum(cnt, 1.0)
    r = jnp.maximum(pooled @ wf1_ref[...] + bf1_ref[...], 0.0)
    out_ref[...] = r @ wf2_ref[...] + bf2_ref[...]


_tcf = pl.pallas_call(
    _tcf_body,
    out_shape=jax.ShapeDtypeStruct((_G, 1), _F32),
)


def kernel(x, edge_index, batch, W1, b1, W2, b2, W3, b3, Wf1, bf1, Wf2, bf2):
    pad = jnp.full((_EPAD - _E,), _N, jnp.int32)
    srcf = jnp.concatenate([edge_index[0], pad])
    dstf = jnp.concatenate([edge_index[1], pad])
    srcw = srcf.reshape(_NW, _NCH, _CHUNK)
    dstw = dstf.reshape(_NW, _NCH, _CHUNK)
    srcp = srcf.reshape(_NW, _EPW // _PCH, _PCH)
    dstp = dstf.reshape(_NW, _EPW // _PCH, _PCH)
    srcs = srcf.reshape(_NW, _EPW // _SCH, _SCH)
    dsts = dstf.reshape(_NW, _EPW // _SCH, _SCH)

    ones_c = jnp.ones((_CHUNK,), _F32)
    zrow1 = jnp.zeros((_RPS,), _F32)
    zrow32 = jnp.zeros((_RPS, _H), _F32)
    zrow64 = jnp.zeros((_RPS, _F // 2), _F32)

    degp = _deg_kernel(dstw, ones_c, zrow1)
    dinv, u0 = _tca(degp.T, x)
    qa = _prop128(u0, srcp, dstp, zrow64)
    s01, u1 = _tcb(qa, dinv, x, W1)
    qb = _prop128(u1, srcp, dstp, zrow64)
    h1, uh1 = _tcc(qb, dinv, x, s01, b1.reshape(1, _H), W1)

    q1 = _prop32(uh1, srcs, dsts, zrow32)
    s01_2, ut1 = _tcd(q1, dinv, h1, W2)
    q2 = _prop32(ut1, srcs, dsts, zrow32)
    h2, uh2 = _tce(q2, dinv, h1, s01_2, b2.reshape(1, _H), W2)

    q3 = _prop32(uh2, srcs, dsts, zrow32)
    s01_3, ut2 = _tcd(q3, dinv, h2, W3)
    q4 = _prop32(ut2, srcs, dsts, zrow32)
    out = _tcf(q4, dinv, h2, s01_3, b3.reshape(1, _H), W3,
               batch.reshape(_N, 1), Wf1, bf1.reshape(1, _H),
               Wf2, bf2.reshape(1, 1))
    return out
